# baseline (speedup 1.0000x reference)
import functools

import numpy as np
import jax
import jax.numpy as jnp
from jax import lax
from jax.experimental import pallas as pl
from jax.experimental.pallas import tpu as pltpu
from jax.experimental.pallas import tpu_sc as plsc

P = 300000
N = 16
F = 3
LANES = 16
POS_FLOATS = N * F

NUM_WORKERS = 32
GROUPS = P // LANES
CG = 16
CHUNK_POS = CG * LANES
CHUNK_FLOATS = CHUNK_POS * POS_FLOATS
TOTAL_CHUNKS = -(-GROUPS // CG)
CHUNKS_PER_WORKER = -(-TOTAL_CHUNKS // NUM_WORKERS)

MAX_DIST = float(np.sqrt(np.float32(19.0) ** 2 + np.float32(19.0) ** 2, dtype=np.float32))
INV_MD = 1.0 / MAX_DIST
DIST_LT_W = 0.5
DIST_LIN_W = 0.5
ANGLE_LT_W = 45.0


@functools.partial(
    pl.kernel,
    mesh=plsc.VectorSubcoreMesh(core_axis_name="c", subcore_axis_name="s"),
    out_type=jax.ShapeDtypeStruct((P,), jnp.float32),
    scratch_types=[
        pltpu.VMEM((CHUNK_FLOATS,), jnp.float32),
        pltpu.VMEM((CHUNK_POS,), jnp.float32),
    ],
    compiler_params=pltpu.CompilerParams(needs_layout_passes=False),
)
def _influences(sda_hbm, out_hbm, in_v, out_v):
    wid = lax.axis_index("s") * 2 + lax.axis_index("c")
    lane = lax.iota(jnp.int32, LANES)
    pos_off = lane * POS_FLOATS

    def chunk_body(k, carry):
        chunk_id = wid + NUM_WORKERS * k
        gstart = jnp.minimum(chunk_id * CG, GROUPS - CG)
        fstart = gstart * (LANES * POS_FLOATS)
        pltpu.sync_copy(sda_hbm.at[pl.ds(fstart, CHUNK_FLOATS)], in_v)

        def group_body(g, gcarry):
            gidx = pos_off + g * (LANES * POS_FLOATS)
            S = [plsc.load_gather(in_v, [gidx + (F * c + 0)]) for c in range(N)]
            A = [plsc.load_gather(in_v, [gidx + (F * c + 2)]) for c in range(N)]
            B = [s < 0.0 for s in S]
            res = jnp.zeros((LANES,), jnp.float32)
            for j in range(N):
                d_j = plsc.load_gather(in_v, [gidx + (F * j + 1)])
                infl = (MAX_DIST - d_j) * INV_MD
                infl = jnp.where(infl < DIST_LT_W, infl * DIST_LIN_W, infl)
                v = infl * S[j]
                for i in range(j):
                    dd = jnp.abs(A[i] - A[j])
                    m = jnp.minimum(dd, 360.0 - dd)
                    q = (m < ANGLE_LT_W) & (B[i] ^ B[j])
                    v = jnp.where(q, v * 0.5, v)
                res = res + v
            out_v[pl.ds(g * LANES, LANES)] = res
            return gcarry

        lax.fori_loop(0, CG, group_body, 0)
        pltpu.sync_copy(out_v, out_hbm.at[pl.ds(gstart * LANES, CHUNK_POS)])
        return carry

    lax.fori_loop(0, CHUNKS_PER_WORKER, chunk_body, 0)


def kernel(stone_dist_angle_input):
    flat = stone_dist_angle_input.reshape(-1)
    return _influences(flat)


# speedup vs baseline: 31.4694x; 31.4694x over previous
import functools

import numpy as np
import jax
import jax.numpy as jnp
from jax import lax
from jax.experimental import pallas as pl
from jax.experimental.pallas import tpu as pltpu
from jax.experimental.pallas import tpu_sc as plsc

P = 300000
N = 16
LANES = 16
ROWS = 3 * N

NUM_WORKERS = 32
CP = 384
GROUPS_PER_CHUNK = CP // LANES
FULL_P = (P // 128) * 128
NUM_CHUNKS = FULL_P // CP
CHUNKS_PER_WORKER = -(-NUM_CHUNKS // NUM_WORKERS)
TAIL_P0 = FULL_P
TAIL = P - FULL_P
TAIL_GROUPS = TAIL // LANES

MAX_DIST = float(np.sqrt(np.float32(19.0) ** 2 + np.float32(19.0) ** 2, dtype=np.float32))
INV_MD = 1.0 / MAX_DIST


def _group_result(S, D, A):
    B = [s < 0.0 for s in S]
    res = jnp.zeros((LANES,), jnp.float32)
    for j in range(N):
        infl = (MAX_DIST - D[j]) * INV_MD
        infl = jnp.where(infl < 0.5, infl * 0.5, infl)
        v = infl * S[j]
        for i in range(j):
            dd = jnp.abs(A[i] - A[j])
            m = jnp.minimum(dd, 360.0 - dd)
            q = (m < 45.0) & (B[i] ^ B[j])
            v = jnp.where(q, v * 0.5, v)
        res = res + v
    return res


@functools.partial(
    pl.kernel,
    mesh=plsc.VectorSubcoreMesh(core_axis_name="c", subcore_axis_name="s"),
    out_type=jax.ShapeDtypeStruct((P,), jnp.float32),
    scratch_types=[
        pltpu.VMEM((ROWS, CP), jnp.float32),
        pltpu.VMEM((CP,), jnp.float32),
        pltpu.VMEM((ROWS, TAIL), jnp.float32),
        pltpu.VMEM((TAIL,), jnp.float32),
    ],
    compiler_params=pltpu.CompilerParams(needs_layout_passes=False),
)
def _influences(x2_hbm, out_hbm, in_v, out_v, tin_v, tout_v):
    wid = lax.axis_index("s") * 2 + lax.axis_index("c")

    def chunk_body(k, carry):
        cid = jnp.minimum(wid + NUM_WORKERS * k, NUM_CHUNKS - 1)
        p0 = cid * CP
        pltpu.sync_copy(x2_hbm.at[:, pl.ds(p0, CP)], in_v)

        def group_body(g, gcarry):
            off = g * LANES
            S = [in_v[c, pl.ds(off, LANES)] for c in range(N)]
            D = [in_v[N + c, pl.ds(off, LANES)] for c in range(N)]
            A = [in_v[2 * N + c, pl.ds(off, LANES)] for c in range(N)]
            out_v[pl.ds(off, LANES)] = _group_result(S, D, A)
            return gcarry

        lax.fori_loop(0, GROUPS_PER_CHUNK, group_body, 0)
        pltpu.sync_copy(out_v, out_hbm.at[pl.ds(p0, CP)])
        return carry

    lax.fori_loop(0, CHUNKS_PER_WORKER, chunk_body, 0)

    @pl.when(wid == 0)
    def _tail():
        pltpu.sync_copy(x2_hbm.at[:, pl.ds(TAIL_P0, TAIL)], tin_v)

        def tail_group(g, gcarry):
            off = g * LANES
            S = [tin_v[c, pl.ds(off, LANES)] for c in range(N)]
            D = [tin_v[N + c, pl.ds(off, LANES)] for c in range(N)]
            A = [tin_v[2 * N + c, pl.ds(off, LANES)] for c in range(N)]
            tout_v[pl.ds(off, LANES)] = _group_result(S, D, A)
            return gcarry

        lax.fori_loop(0, TAIL_GROUPS, tail_group, 0)
        pltpu.sync_copy(tout_v, out_hbm.at[pl.ds(TAIL_P0, TAIL)])


def kernel(stone_dist_angle_input):
    x2 = stone_dist_angle_input.transpose(2, 1, 0).reshape(ROWS, P)
    return _influences(x2)


# speedup vs baseline: 36.0682x; 1.1461x over previous
import functools

import numpy as np
import jax
import jax.numpy as jnp
from jax import lax
from jax.experimental import pallas as pl
from jax.experimental.pallas import tpu as pltpu
from jax.experimental.pallas import tpu_sc as plsc

P = 300000
N = 16
LANES = 16
ROWS = 3 * N

NUM_WORKERS = 32
CP = 384
GROUPS_PER_CHUNK = CP // LANES
FULL_P = (P // 128) * 128
NUM_CHUNKS = FULL_P // CP
CHUNKS_PER_WORKER = 2 * (-(-NUM_CHUNKS // (2 * NUM_WORKERS)))
PAIR_ITERS = CHUNKS_PER_WORKER // 2
TAIL_P0 = FULL_P
TAIL = P - FULL_P
TAIL_GROUPS = TAIL // LANES

MAX_DIST = float(np.sqrt(np.float32(19.0) ** 2 + np.float32(19.0) ** 2, dtype=np.float32))
INV_MD = 1.0 / MAX_DIST


def _group_result(S, D, A):
    B = [s < 0.0 for s in S]
    res = jnp.zeros((LANES,), jnp.float32)
    for j in range(N):
        infl = (MAX_DIST - D[j]) * INV_MD
        infl = jnp.where(infl < 0.5, infl * 0.5, infl)
        v = infl * S[j]
        for i in range(j):
            dd = jnp.abs(A[i] - A[j])
            m = jnp.minimum(dd, 360.0 - dd)
            q = (m < 45.0) & (B[i] ^ B[j])
            v = jnp.where(q, v * 0.5, v)
        res = res + v
    return res


@functools.partial(
    pl.kernel,
    mesh=plsc.VectorSubcoreMesh(core_axis_name="c", subcore_axis_name="s"),
    out_type=jax.ShapeDtypeStruct((P,), jnp.float32),
    scratch_types=[
        pltpu.VMEM((2, ROWS, CP), jnp.float32),
        pltpu.VMEM((CP,), jnp.float32),
        pltpu.VMEM((ROWS, TAIL), jnp.float32),
        pltpu.VMEM((TAIL,), jnp.float32),
        pltpu.SemaphoreType.DMA,
        pltpu.SemaphoreType.DMA,
    ],
    compiler_params=pltpu.CompilerParams(needs_layout_passes=False),
)
def _influences(x2_hbm, out_hbm, in_v, out_v, tin_v, tout_v, sem0, sem1):
    wid = lax.axis_index("s") * 2 + lax.axis_index("c")
    sems = (sem0, sem1)

    def chunk_p0(k):
        return jnp.minimum(wid + NUM_WORKERS * k, NUM_CHUNKS - 1) * CP

    def in_copy(k, b):
        return pltpu.make_async_copy(
            x2_hbm.at[:, pl.ds(chunk_p0(k), CP)], in_v.at[b], sems[b]
        )

    def compute_chunk(p0, b):
        def group_body(g, gcarry):
            off = g * LANES
            S = [in_v[b, c, pl.ds(off, LANES)] for c in range(N)]
            D = [in_v[b, N + c, pl.ds(off, LANES)] for c in range(N)]
            A = [in_v[b, 2 * N + c, pl.ds(off, LANES)] for c in range(N)]
            out_v[pl.ds(off, LANES)] = _group_result(S, D, A)
            return gcarry

        lax.fori_loop(0, GROUPS_PER_CHUNK, group_body, 0)
        pltpu.sync_copy(out_v, out_hbm.at[pl.ds(p0, CP)])

    in_copy(0, 0).start()

    def pair_body(kk, carry):
        k0 = kk * 2
        p0 = chunk_p0(k0)
        in_copy(k0, 0).wait()
        in_copy(k0 + 1, 1).start()
        compute_chunk(p0, 0)

        p1 = chunk_p0(k0 + 1)
        in_copy(k0 + 1, 1).wait()

        @pl.when(kk < PAIR_ITERS - 1)
        def _prefetch():
            in_copy(k0 + 2, 0).start()

        compute_chunk(p1, 1)
        return carry

    lax.fori_loop(0, PAIR_ITERS, pair_body, 0)

    @pl.when(wid == NUM_WORKERS - 1)
    def _tail():
        pltpu.sync_copy(x2_hbm.at[:, pl.ds(TAIL_P0, TAIL)], tin_v)

        def tail_group(g, gcarry):
            off = g * LANES
            S = [tin_v[c, pl.ds(off, LANES)] for c in range(N)]
            D = [tin_v[N + c, pl.ds(off, LANES)] for c in range(N)]
            A = [tin_v[2 * N + c, pl.ds(off, LANES)] for c in range(N)]
            tout_v[pl.ds(off, LANES)] = _group_result(S, D, A)
            return gcarry

        lax.fori_loop(0, TAIL_GROUPS, tail_group, 0)
        pltpu.sync_copy(tout_v, out_hbm.at[pl.ds(TAIL_P0, TAIL)])


def kernel(stone_dist_angle_input):
    x2 = stone_dist_angle_input.transpose(2, 1, 0).reshape(ROWS, P)
    return _influences(x2)


# speedup vs baseline: 37.8614x; 1.0497x over previous
import functools

import numpy as np
import jax
import jax.numpy as jnp
from jax import lax
from jax.experimental import pallas as pl
from jax.experimental.pallas import tpu as pltpu
from jax.experimental.pallas import tpu_sc as plsc

P = 300000
N = 16
LANES = 16
ROWS = 3 * N

NUM_WORKERS = 32
CP = 384
GROUPS_PER_CHUNK = CP // LANES
FULL_P = (P // 128) * 128
NUM_CHUNKS = FULL_P // CP
CHUNKS_PER_WORKER = 2 * (-(-NUM_CHUNKS // (2 * NUM_WORKERS)))
PAIR_ITERS = CHUNKS_PER_WORKER // 2
TAIL_P0 = FULL_P
TAIL = P - FULL_P
TAIL_GROUPS = TAIL // LANES

MAX_DIST = float(np.sqrt(np.float32(19.0) ** 2 + np.float32(19.0) ** 2, dtype=np.float32))
INV_MD = 1.0 / MAX_DIST
ANG_SCALE = float(np.float32(2147483648.0 / 360.0))
C45 = np.uint32(1 << 29)
C90 = np.uint32(1 << 30)


def _group_result(S, D, A):
    W = [
        (((a * ANG_SCALE).astype(jnp.int32) << 1).astype(jnp.uint32)
         | (plsc.bitcast(s, jnp.uint32) >> 31))
        for a, s in zip(A, S)
    ]
    res = jnp.zeros((LANES,), jnp.float32)
    one = np.uint32(1)
    zero = np.uint32(0)
    for j in range(N):
        infl = (MAX_DIST - D[j]) * INV_MD
        infl = jnp.where(infl < 0.5, infl * 0.5, infl)
        v = infl * S[j]
        for i in range(j):
            t = W[i] - W[j]
            q = ((t + C45) < C90) & ((t & one) != zero)
            v = jnp.where(q, v * 0.5, v)
        res = res + v
    return res


@functools.partial(
    pl.kernel,
    mesh=plsc.VectorSubcoreMesh(core_axis_name="c", subcore_axis_name="s"),
    out_type=jax.ShapeDtypeStruct((P,), jnp.float32),
    scratch_types=[
        pltpu.VMEM((2, ROWS, CP), jnp.float32),
        pltpu.VMEM((2, CP), jnp.float32),
        pltpu.VMEM((ROWS, TAIL), jnp.float32),
        pltpu.VMEM((TAIL,), jnp.float32),
        pltpu.SemaphoreType.DMA,
        pltpu.SemaphoreType.DMA,
        pltpu.SemaphoreType.DMA,
        pltpu.SemaphoreType.DMA,
    ],
    compiler_params=pltpu.CompilerParams(needs_layout_passes=False),
)
def _influences(x2_hbm, out_hbm, in_v, out_v, tin_v, tout_v, sem0, sem1, osem0, osem1):
    wid = lax.axis_index("s") * 2 + lax.axis_index("c")
    sems = (sem0, sem1)
    osems = (osem0, osem1)

    def chunk_p0(k):
        return jnp.minimum(wid + NUM_WORKERS * k, NUM_CHUNKS - 1) * CP

    def in_copy(k, b):
        return pltpu.make_async_copy(
            x2_hbm.at[:, pl.ds(chunk_p0(k), CP)], in_v.at[b], sems[b]
        )

    def out_copy(p0, b):
        return pltpu.make_async_copy(
            out_v.at[b], out_hbm.at[pl.ds(p0, CP)], osems[b]
        )

    def compute_chunk(p0, b, kk):
        @pl.when(kk > 0)
        def _wait_prev():
            out_copy(p0, b).wait()

        def group_body(g, gcarry):
            off = g * LANES
            S = [in_v[b, c, pl.ds(off, LANES)] for c in range(N)]
            D = [in_v[b, N + c, pl.ds(off, LANES)] for c in range(N)]
            A = [in_v[b, 2 * N + c, pl.ds(off, LANES)] for c in range(N)]
            out_v[b, pl.ds(off, LANES)] = _group_result(S, D, A)
            return gcarry

        lax.fori_loop(0, GROUPS_PER_CHUNK, group_body, 0)
        out_copy(p0, b).start()

    in_copy(0, 0).start()

    def pair_body(kk, carry):
        k0 = kk * 2
        p0 = chunk_p0(k0)
        in_copy(k0, 0).wait()
        in_copy(k0 + 1, 1).start()
        compute_chunk(p0, 0, kk)

        p1 = chunk_p0(k0 + 1)
        in_copy(k0 + 1, 1).wait()

        @pl.when(kk < PAIR_ITERS - 1)
        def _prefetch():
            in_copy(k0 + 2, 0).start()

        compute_chunk(p1, 1, kk)
        return carry

    lax.fori_loop(0, PAIR_ITERS, pair_body, 0)
    out_copy(chunk_p0(CHUNKS_PER_WORKER - 2), 0).wait()
    out_copy(chunk_p0(CHUNKS_PER_WORKER - 1), 1).wait()

    @pl.when(wid == NUM_WORKERS - 1)
    def _tail():
        pltpu.sync_copy(x2_hbm.at[:, pl.ds(TAIL_P0, TAIL)], tin_v)

        def tail_group(g, gcarry):
            off = g * LANES
            S = [tin_v[c, pl.ds(off, LANES)] for c in range(N)]
            D = [tin_v[N + c, pl.ds(off, LANES)] for c in range(N)]
            A = [tin_v[2 * N + c, pl.ds(off, LANES)] for c in range(N)]
            tout_v[pl.ds(off, LANES)] = _group_result(S, D, A)
            return gcarry

        lax.fori_loop(0, TAIL_GROUPS, tail_group, 0)
        pltpu.sync_copy(tout_v, out_hbm.at[pl.ds(TAIL_P0, TAIL)])


def kernel(stone_dist_angle_input):
    x2 = stone_dist_angle_input.transpose(2, 1, 0).reshape(ROWS, P)
    return _influences(x2)


# speedup vs baseline: 39.2661x; 1.0371x over previous
import functools

import numpy as np
import jax
import jax.numpy as jnp
from jax import lax
from jax.experimental import pallas as pl
from jax.experimental.pallas import tpu as pltpu
from jax.experimental.pallas import tpu_sc as plsc

P = 300000
N = 16
LANES = 16
ROWS = 3 * N

NUM_WORKERS = 32
CP = 384
GROUPS_PER_CHUNK = CP // LANES
FULL_P = (P // 128) * 128
NUM_CHUNKS = FULL_P // CP
BASE_CHUNKS = NUM_CHUNKS // NUM_WORKERS
PAIR_ITERS = BASE_CHUNKS // 2
NUM_EXTRA = NUM_CHUNKS - BASE_CHUNKS * NUM_WORKERS
TAIL_P0 = FULL_P
TAIL = P - FULL_P
TAIL_GROUPS = TAIL // LANES

MAX_DIST = float(np.sqrt(np.float32(19.0) ** 2 + np.float32(19.0) ** 2, dtype=np.float32))
INV_MD = 1.0 / MAX_DIST
ANG_SCALE = float(np.float32(2147483648.0 / 360.0))
C45 = np.uint32(1 << 29)
C90 = np.uint32(1 << 30)


def _group_result(S, D, A):
    W = [
        (((a * ANG_SCALE).astype(jnp.int32) << 1).astype(jnp.uint32)
         | (plsc.bitcast(s, jnp.uint32) >> 31))
        for a, s in zip(A, S)
    ]
    res = jnp.zeros((LANES,), jnp.float32)
    one = np.uint32(1)
    zero = np.uint32(0)
    for j in range(N):
        infl = (MAX_DIST - D[j]) * INV_MD
        infl = jnp.where(infl < 0.5, infl * 0.5, infl)
        v = infl * S[j]
        for i in range(j):
            t = W[i] - W[j]
            q = ((t + C45) < C90) & ((t & one) != zero)
            v = jnp.where(q, v * 0.5, v)
        res = res + v
    return res


@functools.partial(
    pl.kernel,
    mesh=plsc.VectorSubcoreMesh(core_axis_name="c", subcore_axis_name="s"),
    out_type=jax.ShapeDtypeStruct((P,), jnp.float32),
    scratch_types=[
        pltpu.VMEM((2, ROWS, CP), jnp.float32),
        pltpu.VMEM((2, CP), jnp.float32),
        pltpu.VMEM((ROWS, TAIL), jnp.float32),
        pltpu.VMEM((TAIL,), jnp.float32),
        pltpu.SemaphoreType.DMA,
        pltpu.SemaphoreType.DMA,
        pltpu.SemaphoreType.DMA,
        pltpu.SemaphoreType.DMA,
    ],
    compiler_params=pltpu.CompilerParams(needs_layout_passes=False),
)
def _influences(x2_hbm, out_hbm, in_v, out_v, tin_v, tout_v, sem0, sem1, osem0, osem1):
    wid = lax.axis_index("s") * 2 + lax.axis_index("c")
    sems = (sem0, sem1)
    osems = (osem0, osem1)

    def chunk_p0(k):
        return jnp.minimum(wid + NUM_WORKERS * k, NUM_CHUNKS - 1) * CP

    def in_copy(k, b):
        return pltpu.make_async_copy(
            x2_hbm.at[:, pl.ds(chunk_p0(k), CP)], in_v.at[b], sems[b]
        )

    def out_copy(p0, b):
        return pltpu.make_async_copy(
            out_v.at[b], out_hbm.at[pl.ds(p0, CP)], osems[b]
        )

    def compute_chunk(p0, b, kk):
        @pl.when(kk > 0)
        def _wait_prev():
            out_copy(p0, b).wait()

        def group_body(g, gcarry):
            off = g * LANES
            S = [in_v[b, c, pl.ds(off, LANES)] for c in range(N)]
            D = [in_v[b, N + c, pl.ds(off, LANES)] for c in range(N)]
            A = [in_v[b, 2 * N + c, pl.ds(off, LANES)] for c in range(N)]
            out_v[b, pl.ds(off, LANES)] = _group_result(S, D, A)
            return gcarry

        lax.fori_loop(0, GROUPS_PER_CHUNK, group_body, 0)
        out_copy(p0, b).start()

    in_copy(0, 0).start()

    def pair_body(kk, carry):
        k0 = kk * 2
        p0 = chunk_p0(k0)
        in_copy(k0, 0).wait()
        in_copy(k0 + 1, 1).start()
        compute_chunk(p0, 0, kk)

        p1 = chunk_p0(k0 + 1)
        in_copy(k0 + 1, 1).wait()

        @pl.when((kk < PAIR_ITERS - 1) | ((kk == PAIR_ITERS - 1) & (wid < NUM_EXTRA)))
        def _prefetch():
            in_copy(k0 + 2, 0).start()

        compute_chunk(p1, 1, kk)
        return carry

    lax.fori_loop(0, PAIR_ITERS, pair_body, 0)

    @pl.when(wid < NUM_EXTRA)
    def _extra():
        in_copy(BASE_CHUNKS, 0).wait()
        compute_chunk(chunk_p0(BASE_CHUNKS), 0, 1)

    out_copy(chunk_p0(0), 0).wait()
    out_copy(chunk_p0(1), 1).wait()

    @pl.when(wid == NUM_WORKERS - 1)
    def _tail():
        pltpu.sync_copy(x2_hbm.at[:, pl.ds(TAIL_P0, TAIL)], tin_v)

        def tail_group(g, gcarry):
            off = g * LANES
            S = [tin_v[c, pl.ds(off, LANES)] for c in range(N)]
            D = [tin_v[N + c, pl.ds(off, LANES)] for c in range(N)]
            A = [tin_v[2 * N + c, pl.ds(off, LANES)] for c in range(N)]
            tout_v[pl.ds(off, LANES)] = _group_result(S, D, A)
            return gcarry

        lax.fori_loop(0, TAIL_GROUPS, tail_group, 0)
        pltpu.sync_copy(tout_v, out_hbm.at[pl.ds(TAIL_P0, TAIL)])


def kernel(stone_dist_angle_input):
    x2 = stone_dist_angle_input.transpose(2, 1, 0).reshape(ROWS, P)
    return _influences(x2)


# speedup vs baseline: 50.9076x; 1.2965x over previous
import functools

import numpy as np
import jax
import jax.numpy as jnp
from jax import lax
from jax.experimental import pallas as pl
from jax.experimental.pallas import tpu as pltpu
from jax.experimental.pallas import tpu_sc as plsc

P = 300000
N = 16
LANES = 16
ROWS = 3 * N

NUM_WORKERS = 32
CP = 384
GROUPS_PER_CHUNK = CP // LANES
FULL_P = (P // 128) * 128
NUM_CHUNKS = FULL_P // CP
BASE_CHUNKS = NUM_CHUNKS // NUM_WORKERS
PAIR_ITERS = BASE_CHUNKS // 2
NUM_EXTRA = NUM_CHUNKS - BASE_CHUNKS * NUM_WORKERS
TAIL_P0 = FULL_P
TAIL = P - FULL_P
TAIL_GROUPS = TAIL // LANES

MAX_DIST = float(np.sqrt(np.float32(19.0) ** 2 + np.float32(19.0) ** 2, dtype=np.float32))
INV_MD = 1.0 / MAX_DIST
ANG_SCALE = float(np.float32(2147483648.0 / 360.0))
C45 = np.uint32(1 << 29)
QMASK = np.uint32(0xC0000001)
QVAL = np.uint32(1)


def _group_result(row):
    WC = []
    for c in range(N):
        a = row(2 * N + c)
        s = row(c)
        w = ((a * ANG_SCALE).astype(jnp.int32) << 1).astype(jnp.uint32)
        WC.append((w | (plsc.bitcast(s, jnp.uint32) >> 31)) + C45)
    res = jnp.zeros((LANES,), jnp.float32)
    for j in range(N):
        infl = (MAX_DIST - row(N + j)) * INV_MD
        infl = jnp.where(infl < 0.5, infl * 0.5, infl)
        v = infl * row(j)
        wj = WC[j] - C45
        for i in range(j):
            q = ((WC[i] - wj) & QMASK) == QVAL
            v = jnp.where(q, v * 0.5, v)
        res = res + v
    return res


@functools.partial(
    pl.kernel,
    mesh=plsc.VectorSubcoreMesh(core_axis_name="c", subcore_axis_name="s"),
    out_type=jax.ShapeDtypeStruct((P,), jnp.float32),
    scratch_types=[
        pltpu.VMEM((2, ROWS, CP), jnp.float32),
        pltpu.VMEM((2, CP), jnp.float32),
        pltpu.VMEM((ROWS, TAIL), jnp.float32),
        pltpu.VMEM((TAIL,), jnp.float32),
        pltpu.SemaphoreType.DMA,
        pltpu.SemaphoreType.DMA,
        pltpu.SemaphoreType.DMA,
        pltpu.SemaphoreType.DMA,
    ],
    compiler_params=pltpu.CompilerParams(needs_layout_passes=False),
)
def _influences(x2_hbm, out_hbm, in_v, out_v, tin_v, tout_v, sem0, sem1, osem0, osem1):
    wid = lax.axis_index("s") * 2 + lax.axis_index("c")
    sems = (sem0, sem1)
    osems = (osem0, osem1)

    def chunk_p0(k):
        return jnp.minimum(wid + NUM_WORKERS * k, NUM_CHUNKS - 1) * CP

    def in_copy(k, b):
        return pltpu.make_async_copy(
            x2_hbm.at[:, pl.ds(chunk_p0(k), CP)], in_v.at[b], sems[b]
        )

    def out_copy(p0, b):
        return pltpu.make_async_copy(
            out_v.at[b], out_hbm.at[pl.ds(p0, CP)], osems[b]
        )

    def compute_chunk(p0, b, kk):
        @pl.when(kk > 0)
        def _wait_prev():
            out_copy(p0, b).wait()

        def group_body(g, gcarry):
            off = g * LANES
            row = lambda r: in_v[b, r, pl.ds(off, LANES)]
            out_v[b, pl.ds(off, LANES)] = _group_result(row)
            return gcarry

        lax.fori_loop(0, GROUPS_PER_CHUNK, group_body, 0)
        out_copy(p0, b).start()

    in_copy(0, 0).start()

    def pair_body(kk, carry):
        k0 = kk * 2
        p0 = chunk_p0(k0)
        in_copy(k0, 0).wait()
        in_copy(k0 + 1, 1).start()
        compute_chunk(p0, 0, kk)

        p1 = chunk_p0(k0 + 1)
        in_copy(k0 + 1, 1).wait()

        @pl.when((kk < PAIR_ITERS - 1) | ((kk == PAIR_ITERS - 1) & (wid < NUM_EXTRA)))
        def _prefetch():
            in_copy(k0 + 2, 0).start()

        compute_chunk(p1, 1, kk)
        return carry

    lax.fori_loop(0, PAIR_ITERS, pair_body, 0)

    @pl.when(wid < NUM_EXTRA)
    def _extra():
        in_copy(BASE_CHUNKS, 0).wait()
        compute_chunk(chunk_p0(BASE_CHUNKS), 0, 1)

    out_copy(chunk_p0(0), 0).wait()
    out_copy(chunk_p0(1), 1).wait()

    @pl.when(wid == NUM_WORKERS - 1)
    def _tail():
        pltpu.sync_copy(x2_hbm.at[:, pl.ds(TAIL_P0, TAIL)], tin_v)

        def tail_group(g, gcarry):
            off = g * LANES
            row = lambda r: tin_v[r, pl.ds(off, LANES)]
            tout_v[pl.ds(off, LANES)] = _group_result(row)
            return gcarry

        lax.fori_loop(0, TAIL_GROUPS, tail_group, 0)
        pltpu.sync_copy(tout_v, out_hbm.at[pl.ds(TAIL_P0, TAIL)])


def kernel(stone_dist_angle_input):
    x2 = stone_dist_angle_input.transpose(2, 1, 0).reshape(ROWS, P)
    return _influences(x2)
